# Initial kernel scaffold; baseline (speedup 1.0000x reference)
import jax
import jax.numpy as jnp
from jax.experimental import pallas as pl


def kernel(Z, pos, shift_vecs, cell_volume, c6ab, r0ab, rcov, r2r4):
    raise NotImplementedError("write your pallas kernel here")


# speedup vs baseline: 46.9177x; 46.9177x over previous
import functools

import jax
import jax.numpy as jnp
from jax.experimental import pallas as pl
from jax.experimental.pallas import tpu as pltpu

BOHR = 0.52917721067
D3_AUTOEV = 27.21138505
D3_K1 = 16.0
D3_K3 = 4.0

_N = 320
_S = 27
_ZP = 128

_S6, _RS6, _S18, _RS18, _ALP = 1.0, 1.217, 0.722, 1.0, 14.0
_CUTOFF = 95.0
_CNTHR = 40.0

_HI = jax.lax.Precision.HIGHEST


def _dot(a, b):
    return jax.lax.dot_general(a, b, (((1,), (0,)), ((), ())),
                               precision=_HI, preferred_element_type=jnp.float32)


def _body(o_ref, ot_ref, posb_ref, post_ref, shb_ref,
          c6r_ref, cni_ref, cnj_ref, r0ab_ref, rcovc_ref, rcovr_ref,
          r4c_ref, r4r_ref,
          out_ref,
          pc_ref, pr_ref, ss2_ref, rall_ref, r20_ref, rcij_ref,
          r0s_ref, c6s_ref, c8s_ref, wsum_ref, wc6_ref):
    O = o_ref[...]
    OT = ot_ref[...]
    posb = posb_ref[...]
    posT = post_ref[...]
    shb = shb_ref[...]

    pc_ref[...] = jnp.sum(shb[:, None, :] * posb[None, :, :], axis=-1,
                          keepdims=True)
    pr_ref[...] = jnp.sum(shb[:, :, None] * posT[None, :, :], axis=1,
                          keepdims=True)
    ss2_ref[...] = jnp.sum(shb * shb, axis=-1, keepdims=True)[:, :, None]

    d0 = posb[:, 0:1] - posT[0:1, :]
    d1 = posb[:, 1:2] - posT[1:2, :]
    d2 = posb[:, 2:3] - posT[2:3, :]
    r20_ref[...] = d0 * d0 + d1 * d1 + d2 * d2

    rc_col = _dot(O, rcovc_ref[...])
    rc_row = _dot(rcovr_ref[...], OT)
    rcij_ref[...] = rc_col + rc_row
    r0s_ref[...] = _dot(_dot(O, r0ab_ref[...]), OT)

    def cn_step(s, carry):
        cnc, cnr = carry
        r2 = (r20_ref[...] + 2.0 * pc_ref[s] - 2.0 * pr_ref[s] + ss2_ref[s])
        rs = jnp.sqrt(jnp.where(r2 > 1e-8, r2, 1e12))
        rall_ref[s] = rs
        damp = 1.0 / (1.0 + jnp.exp(-D3_K1 * (rcij_ref[...] / rs - 1.0)))
        c = jnp.where(rs <= _CNTHR, damp, 0.0)
        return (cnc + jnp.sum(c, axis=1, keepdims=True),
                cnr + jnp.sum(c, axis=0, keepdims=True))

    cn_col, cn_row = jax.lax.fori_loop(
        0, _S, cn_step,
        (jnp.zeros((_N, 1), jnp.float32), jnp.zeros((1, _N), jnp.float32)))

    wsum_ref[...] = jnp.zeros((_N, _N), jnp.float32)
    wc6_ref[...] = jnp.zeros((_N, _N), jnp.float32)

    def c6_step(m, _):
        c6r = _dot(_dot(O, c6r_ref[m]), OT)
        cni = _dot(_dot(O, cni_ref[m]), OT)
        cnj = _dot(_dot(O, cnj_ref[m]), OT)
        di = cn_col - cni
        dj = cn_row - cnj
        w = jnp.exp(-D3_K3 * (di * di + dj * dj))
        wsum_ref[...] += w
        wc6_ref[...] += w * c6r
        return 0

    jax.lax.fori_loop(0, 25, c6_step, 0)

    c6 = wc6_ref[...] / (wsum_ref[...] + 1e-12)
    c6s_ref[...] = c6
    r4c = _dot(O, r4c_ref[...])
    r4r = _dot(r4r_ref[...], OT)
    c8s_ref[...] = 3.0 * c6 * r4c * r4r

    def e_step(s, esum):
        rs = rall_ref[s]
        rr = r0s_ref[...] / rs
        x = _RS6 * rr
        x2 = x * x
        x4 = x2 * x2
        x8 = x4 * x4
        damp6 = 1.0 / (1.0 + 6.0 * (x8 * x4 * x2))
        y2 = rr * rr
        y4 = y2 * y2
        y8 = y4 * y4
        damp8 = 1.0 / (1.0 + 6.0 * (y8 * y8))
        r2v = rs * rs
        r6 = r2v * r2v * r2v
        r8 = r6 * r2v
        term = (_S6 * c6s_ref[...] / r6) * damp6 + \
               (_S18 * c8s_ref[...] / r8) * damp8
        return esum + jnp.sum(jnp.where(rs <= _CUTOFF, term, 0.0))

    esum = jax.lax.fori_loop(0, _S, e_step, jnp.float32(0.0))
    out_ref[...] = (-0.5 * D3_AUTOEV * esum).reshape(1, 1)


_SCRATCH = [
    pltpu.VMEM((_S, _N, 1), jnp.float32),
    pltpu.VMEM((_S, 1, _N), jnp.float32),
    pltpu.VMEM((_S, 1, 1), jnp.float32),
    pltpu.VMEM((_S, _N, _N), jnp.float32),
    pltpu.VMEM((_N, _N), jnp.float32),
    pltpu.VMEM((_N, _N), jnp.float32),
    pltpu.VMEM((_N, _N), jnp.float32),
    pltpu.VMEM((_N, _N), jnp.float32),
    pltpu.VMEM((_N, _N), jnp.float32),
    pltpu.VMEM((_N, _N), jnp.float32),
    pltpu.VMEM((_N, _N), jnp.float32),
]


@functools.partial(jax.jit, static_argnums=())
def kernel(Z, pos, shift_vecs, cell_volume, c6ab, r0ab, rcov, r2r4):
    f32 = jnp.float32
    Zc = jnp.clip(Z, 0, 94).astype(jnp.int32)
    O = (Zc[:, None] == jnp.arange(_ZP, dtype=jnp.int32)[None, :]).astype(f32)
    posb = (pos / BOHR).astype(f32)
    shb = (shift_vecs / BOHR).astype(f32)

    tbl = jnp.transpose(c6ab, (2, 3, 0, 1, 4)).reshape(25, 95, 95, 3)
    tbl = jnp.pad(tbl, ((0, 0), (0, _ZP - 95), (0, _ZP - 95), (0, 0)))
    c6r_t, cni_t, cnj_t = tbl[..., 0], tbl[..., 1], tbl[..., 2]
    r0p = jnp.pad(r0ab, ((0, _ZP - 95), (0, _ZP - 95)))
    rcp = jnp.pad(rcov, (0, _ZP - 95))
    r4p = jnp.pad(r2r4, (0, _ZP - 95))

    out = pl.pallas_call(
        _body,
        out_shape=jax.ShapeDtypeStruct((1, 1), f32),
        scratch_shapes=_SCRATCH,
    )(O, O.T, posb, posb.T, shb,
      c6r_t, cni_t, cnj_t, r0p, rcp[:, None], rcp[None, :],
      r4p[:, None], r4p[None, :])
    return out[0, 0]


# speedup vs baseline: 62.0004x; 1.3215x over previous
import functools

import jax
import jax.numpy as jnp
from jax.experimental import pallas as pl
from jax.experimental.pallas import tpu as pltpu

BOHR = 0.52917721067
D3_AUTOEV = 27.21138505
D3_K1 = 16.0
D3_K3 = 4.0

_N = 320
_S = 27
_ZP = 128

_S6, _RS6, _S18, _RS18, _ALP = 1.0, 1.217, 0.722, 1.0, 14.0
_CUTOFF = 95.0
_CNTHR = 40.0

def _dot(a, b, precision=jax.lax.Precision.HIGHEST):
    return jax.lax.dot_general(a, b, (((1,), (0,)), ((), ())),
                               precision=precision,
                               preferred_element_type=jnp.float32)


def _split_bf16(x):
    hi = x.astype(jnp.bfloat16)
    lo = (x - hi.astype(jnp.float32)).astype(jnp.bfloat16)
    return hi, lo


def _body(o_ref, ot_ref, obf_ref, otbf_ref, posb_ref, post_ref, shb_ref,
          tbl_hi_ref, tbl_lo_ref, r0ab_ref, rcovc_ref, rcovr_ref,
          r4c_ref, r4r_ref,
          out_ref,
          pc_ref, pr_ref, ss2_ref, rall_ref, r20_ref, rcij_ref,
          r0s_ref, c6s_ref, c8s_ref, wsum_ref, wc6_ref):
    O = o_ref[...]
    OT = ot_ref[...]
    Obf = obf_ref[...]
    OTbf = otbf_ref[...]
    posb = posb_ref[...]
    posT = post_ref[...]
    shb = shb_ref[...]

    pc_ref[...] = jnp.sum(shb[:, None, :] * posb[None, :, :], axis=-1,
                          keepdims=True)
    pr_ref[...] = jnp.sum(shb[:, :, None] * posT[None, :, :], axis=1,
                          keepdims=True)
    ss2_ref[...] = jnp.sum(shb * shb, axis=-1, keepdims=True)[:, :, None]

    d0 = posb[:, 0:1] - posT[0:1, :]
    d1 = posb[:, 1:2] - posT[1:2, :]
    d2 = posb[:, 2:3] - posT[2:3, :]
    r20_ref[...] = d0 * d0 + d1 * d1 + d2 * d2

    rc_col = _dot(O, rcovc_ref[...])
    rc_row = _dot(rcovr_ref[...], OT)
    rcij_ref[...] = rc_col + rc_row
    r0s_ref[...] = _dot(_dot(O, r0ab_ref[...]), OT)

    def cn_step(s, carry):
        cnc, cnr = carry
        r2 = (r20_ref[...] + 2.0 * pc_ref[s] - 2.0 * pr_ref[s] + ss2_ref[s])
        ri = jax.lax.rsqrt(jnp.where(r2 > 1e-8, r2, 1e12))
        rall_ref[s] = ri
        damp = 1.0 / (1.0 + jnp.exp(-D3_K1 * (rcij_ref[...] * ri - 1.0)))
        c = jnp.where(ri >= 1.0 / _CNTHR, damp, 0.0)
        return (cnc + jnp.sum(c, axis=1, keepdims=True),
                cnr + jnp.sum(c, axis=0, keepdims=True))

    cn_col, cn_row = jax.lax.fori_loop(
        0, _S, cn_step,
        (jnp.zeros((_N, 1), jnp.float32), jnp.zeros((1, _N), jnp.float32)))

    wsum_ref[...] = jnp.zeros((_N, _N), jnp.float32)
    wc6_ref[...] = jnp.zeros((_N, _N), jnp.float32)

    dflt = jax.lax.Precision.DEFAULT

    def c6_step(m, _):
        g = _dot(Obf, tbl_hi_ref[m], dflt) + _dot(Obf, tbl_lo_ref[m], dflt)
        gs = jnp.concatenate([g[:, 0:_ZP], g[:, _ZP:2 * _ZP],
                              g[:, 2 * _ZP:3 * _ZP]], axis=0)
        gh, gl = _split_bf16(gs)
        r = _dot(gh, OTbf, dflt) + _dot(gl, OTbf, dflt)
        c6r = r[0:_N]
        cni = r[_N:2 * _N]
        cnj = r[2 * _N:3 * _N]
        di = cn_col - cni
        dj = cn_row - cnj
        w = jnp.exp(-D3_K3 * (di * di + dj * dj))
        wsum_ref[...] += w
        wc6_ref[...] += w * c6r
        return 0

    jax.lax.fori_loop(0, 25, c6_step, 0)

    c6 = wc6_ref[...] / (wsum_ref[...] + 1e-12)
    c6s_ref[...] = c6
    r4c = _dot(O, r4c_ref[...])
    r4r = _dot(r4r_ref[...], OT)
    c8s_ref[...] = 3.0 * c6 * r4c * r4r

    def e_step(s, esum):
        ri = rall_ref[s]
        rr = r0s_ref[...] * ri
        x = _RS6 * rr
        x2 = x * x
        x4 = x2 * x2
        x8 = x4 * x4
        damp6 = 1.0 / (1.0 + 6.0 * (x8 * x4 * x2))
        y2 = rr * rr
        y4 = y2 * y2
        y8 = y4 * y4
        damp8 = 1.0 / (1.0 + 6.0 * (y8 * y8))
        ri2 = ri * ri
        ri6 = ri2 * ri2 * ri2
        ri8 = ri6 * ri2
        term = (_S6 * c6s_ref[...] * ri6) * damp6 + \
               (_S18 * c8s_ref[...] * ri8) * damp8
        return esum + jnp.sum(jnp.where(ri >= 1.0 / _CUTOFF, term, 0.0))

    esum = jax.lax.fori_loop(0, _S, e_step, jnp.float32(0.0))
    out_ref[...] = (-0.5 * D3_AUTOEV * esum).reshape(1, 1)


_SCRATCH = [
    pltpu.VMEM((_S, _N, 1), jnp.float32),
    pltpu.VMEM((_S, 1, _N), jnp.float32),
    pltpu.VMEM((_S, 1, 1), jnp.float32),
    pltpu.VMEM((_S, _N, _N), jnp.float32),
    pltpu.VMEM((_N, _N), jnp.float32),
    pltpu.VMEM((_N, _N), jnp.float32),
    pltpu.VMEM((_N, _N), jnp.float32),
    pltpu.VMEM((_N, _N), jnp.float32),
    pltpu.VMEM((_N, _N), jnp.float32),
    pltpu.VMEM((_N, _N), jnp.float32),
    pltpu.VMEM((_N, _N), jnp.float32),
]


@functools.partial(jax.jit, static_argnums=())
def kernel(Z, pos, shift_vecs, cell_volume, c6ab, r0ab, rcov, r2r4):
    f32 = jnp.float32
    Zc = jnp.clip(Z, 0, 94).astype(jnp.int32)
    O = (Zc[:, None] == jnp.arange(_ZP, dtype=jnp.int32)[None, :]).astype(f32)
    posb = (pos / BOHR).astype(f32)
    shb = (shift_vecs / BOHR).astype(f32)

    tbl = jnp.transpose(c6ab, (2, 3, 0, 1, 4)).reshape(25, 95, 95, 3)
    tbl = jnp.pad(tbl, ((0, 0), (0, _ZP - 95), (0, _ZP - 95), (0, 0)))
    tbl = jnp.transpose(tbl, (0, 1, 3, 2)).reshape(25, _ZP, 3 * _ZP)
    tbl_hi = tbl.astype(jnp.bfloat16)
    tbl_lo = (tbl - tbl_hi.astype(f32)).astype(jnp.bfloat16)
    r0p = jnp.pad(r0ab, ((0, _ZP - 95), (0, _ZP - 95)))
    rcp = jnp.pad(rcov, (0, _ZP - 95))
    r4p = jnp.pad(r2r4, (0, _ZP - 95))

    Obf = O.astype(jnp.bfloat16)
    out = pl.pallas_call(
        _body,
        out_shape=jax.ShapeDtypeStruct((1, 1), f32),
        scratch_shapes=_SCRATCH,
    )(O, O.T, Obf, Obf.T, posb, posb.T, shb,
      tbl_hi, tbl_lo, r0p, rcp[:, None], rcp[None, :],
      r4p[:, None], r4p[None, :])
    return out[0, 0]
